# baseline (speedup 1.0000x reference)
import functools

import jax
import jax.numpy as jnp
from jax import lax
from jax.experimental import pallas as pl
from jax.experimental.pallas import tpu as pltpu
from jax.experimental.pallas import tpu_sc as plsc


@functools.lru_cache(maxsize=None)
def _make(B, T, V, D):
    info = plsc.get_sparse_core_info()
    NC, NS = info.num_cores, info.num_subcores
    NW = NC * NS
    assert B % (128 * NW) == 0 and T % 8 == 0 and D == 64
    n_t = T

    mesh = plsc.VectorSubcoreMesh(core_axis_name="c", subcore_axis_name="s")

    scratch = [
        pltpu.VMEM((8, 128), jnp.int32),
        pltpu.VMEM((128,), jnp.int32),
        pltpu.VMEM((128,), jnp.int32),
        pltpu.VMEM((128, 128), jnp.float32),
        pltpu.VMEM((128, 128), jnp.float32),
        pltpu.VMEM((64, 128), jnp.float32),
        pltpu.VMEM((64, 128), jnp.float32),
        pltpu.SemaphoreType.DMA,
        pltpu.SemaphoreType.DMA,
        pltpu.SemaphoreType.DMA,
        pltpu.SemaphoreType.DMA,
    ]

    @functools.partial(
        pl.kernel,
        mesh=mesh,
        out_type=jax.ShapeDtypeStruct((T, D, B), jnp.float32),
        scratch_types=scratch,
        compiler_params=pltpu.CompilerParams(needs_layout_passes=False),
    )
    def k(tok_hbm, table_hbm, out_hbm, tokv, pidx0, pidx1, gbuf0, gbuf1,
          tbuf0, tbuf1, sg0, sg1, so0, so1):
        pidx = (pidx0, pidx1)
        gbuf = (gbuf0, gbuf1)
        tbuf = (tbuf0, tbuf1)
        sg = (sg0, sg1)
        so = (so0, so1)
        c = lax.axis_index("s") * NC + lax.axis_index("c")
        col0 = c * 128
        iota16 = jax.lax.iota(jnp.int32, 16)

        def gather_desc(u, ib):
            return pltpu.make_async_copy(
                table_hbm.at[pidx[ib]], gbuf[ib], sg[ib])

        def write_desc(tp, ib):
            return pltpu.make_async_copy(
                tbuf[ib], out_hbm.at[tp, pl.ds(0, D), pl.ds(col0, 128)],
                so[ib])

        def transpose_and_write(tp, ib):
            tp8 = lax.rem(tp, 8)
            for g in range(8):
                tv = tokv[tp8, pl.ds(g * 16, 16)]
                parv = lax.shift_left(
                    lax.bitwise_and(tv, jnp.int32(1)), jnp.int32(6))
                rowv = iota16 + jnp.int32(g * 16)
                @plsc.parallel_loop(0, 16, 1, unroll=8)
                def _diag(j):
                    mj = lax.bitwise_and(iota16 + j, jnp.int32(15))
                    cj = parv + mj
                    for d0 in range(0, D, 16):
                        dj = mj + jnp.int32(d0)
                        vals = plsc.load_gather(
                            gbuf[ib], [rowv, cj + jnp.int32(d0)])
                        plsc.store_scatter(tbuf[ib], [dj, rowv], vals)
            write_desc(tp, ib).start()

        def stage_and_gather(t, u):
            @pl.when(lax.rem(t, 8) == 0)
            def _stage():
                pltpu.sync_copy(
                    tok_hbm.at[pl.ds((t // 8) * 8, 8), pl.ds(col0, 128)],
                    tokv)
            t8 = lax.rem(t, 8)
            for g in range(8):
                tv = tokv[t8, pl.ds(g * 16, 16)]
                pidx[u][pl.ds(g * 16, 16)] = lax.shift_right_logical(
                    tv, jnp.int32(1))
            gather_desc(u, u).start()

        def body(i, carry):
            for u in (0, 1):
                t = i * 2 + u
                bp = 1 - u

                @pl.when(t > 0)
                def _finish_prev():
                    gather_desc(t - 1, bp).wait()

                    @pl.when(t > 2)
                    def _drain_write():
                        write_desc(t - 3, bp).wait()
                    transpose_and_write(t - 1, bp)

                stage_and_gather(t, u)
            return carry

        lax.fori_loop(0, n_t // 2, body, 0)

        gather_desc(n_t - 1, 1).wait()
        write_desc(n_t - 3, 1).wait()
        transpose_and_write(n_t - 1, 1)
        write_desc(n_t - 2, 0).wait()
        write_desc(n_t - 1, 1).wait()

    return k


def kernel(tokens, embeddings):
    B, T = tokens.shape
    V, D = embeddings.shape
    tok_t = tokens.T
    emb128 = embeddings.reshape(V // 2, 2 * D)
    out_t = _make(B, T, V, D)(tok_t, emb128)
    return out_t.transpose(2, 0, 1)


# speedup vs baseline: 1.0210x; 1.0210x over previous
import functools

import jax
import jax.numpy as jnp
from jax import lax
from jax.experimental import pallas as pl
from jax.experimental.pallas import tpu as pltpu
from jax.experimental.pallas import tpu_sc as plsc


@functools.lru_cache(maxsize=None)
def _make(B, T, V, D):
    info = plsc.get_sparse_core_info()
    NC, NS = info.num_cores, info.num_subcores
    NW = NC * NS
    assert B % (128 * NW) == 0 and T % 8 == 0 and D == 64
    n_t = T

    mesh = plsc.VectorSubcoreMesh(core_axis_name="c", subcore_axis_name="s")

    scratch = [
        pltpu.VMEM((8, 128), jnp.int32),
        pltpu.VMEM((128,), jnp.int32),
        pltpu.VMEM((128,), jnp.int32),
        pltpu.VMEM((128, 128), jnp.float32),
        pltpu.VMEM((128, 128), jnp.float32),
        pltpu.VMEM((64, 128), jnp.float32),
        pltpu.VMEM((64, 128), jnp.float32),
        pltpu.SemaphoreType.DMA,
        pltpu.SemaphoreType.DMA,
        pltpu.SemaphoreType.DMA,
        pltpu.SemaphoreType.DMA,
    ]

    @functools.partial(
        pl.kernel,
        mesh=mesh,
        out_type=jax.ShapeDtypeStruct((T, D, B), jnp.float32),
        scratch_types=scratch,
        compiler_params=pltpu.CompilerParams(needs_layout_passes=False),
    )
    def k(tok_hbm, table_hbm, out_hbm, tokv, pidx0, pidx1, gbuf0, gbuf1,
          tbuf0, tbuf1, sg0, sg1, so0, so1):
        pidx = (pidx0, pidx1)
        gbuf = (gbuf0, gbuf1)
        tbuf = (tbuf0, tbuf1)
        sg = (sg0, sg1)
        so = (so0, so1)
        c = lax.axis_index("s") * NC + lax.axis_index("c")
        col0 = c * 128
        iota16 = jax.lax.iota(jnp.int32, 16)

        def gather_desc(u, ib):
            return pltpu.make_async_copy(
                table_hbm.at[pidx[ib]], gbuf[ib], sg[ib])

        def write_desc(tp, ib):
            return pltpu.make_async_copy(
                tbuf[ib], out_hbm.at[tp, pl.ds(0, D), pl.ds(col0, 128)],
                so[ib])

        def transpose_and_write(tp, ib):
            tp8 = lax.rem(tp, 8)
            for g in range(8):
                tv = tokv[tp8, pl.ds(g * 16, 16)]
                parv = lax.shift_left(
                    lax.bitwise_and(tv, jnp.int32(1)), jnp.int32(6))
                rowv = iota16 + jnp.int32(g * 16)
                @plsc.parallel_loop(0, 16, 1, unroll=4)
                def _diag(j):
                    mj = lax.bitwise_and(iota16 + j, jnp.int32(15))
                    cj = parv + mj
                    for d0 in range(0, D, 16):
                        dj = mj + jnp.int32(d0)
                        vals = plsc.load_gather(
                            gbuf[ib], [rowv, cj + jnp.int32(d0)])
                        plsc.store_scatter(tbuf[ib], [dj, rowv], vals)
            write_desc(tp, ib).start()

        def stage_and_gather(t, u):
            @pl.when(lax.rem(t, 8) == 0)
            def _stage():
                pltpu.sync_copy(
                    tok_hbm.at[pl.ds((t // 8) * 8, 8), pl.ds(col0, 128)],
                    tokv)
            t8 = lax.rem(t, 8)
            for g in range(8):
                tv = tokv[t8, pl.ds(g * 16, 16)]
                pidx[u][pl.ds(g * 16, 16)] = lax.shift_right_logical(
                    tv, jnp.int32(1))
            gather_desc(u, u).start()

        def body(i, carry):
            for u in (0, 1):
                t = i * 2 + u
                bp = 1 - u

                @pl.when(t > 0)
                def _finish_prev():
                    gather_desc(t - 1, bp).wait()

                    @pl.when(t > 2)
                    def _drain_write():
                        write_desc(t - 3, bp).wait()
                    transpose_and_write(t - 1, bp)

                stage_and_gather(t, u)
            return carry

        lax.fori_loop(0, n_t // 2, body, 0)

        gather_desc(n_t - 1, 1).wait()
        write_desc(n_t - 3, 1).wait()
        transpose_and_write(n_t - 1, 1)
        write_desc(n_t - 2, 0).wait()
        write_desc(n_t - 1, 1).wait()

    return k


def kernel(tokens, embeddings):
    B, T = tokens.shape
    V, D = embeddings.shape
    tok_t = tokens.T
    emb128 = embeddings.reshape(V // 2, 2 * D)
    out_t = _make(B, T, V, D)(tok_t, emb128)
    return out_t.transpose(2, 0, 1)
